# Initial kernel scaffold; baseline (speedup 1.0000x reference)
import jax
import jax.numpy as jnp
from jax.experimental import pallas as pl


def kernel(V_no_pos, V_pos, R_s, R_r, W1, b1, W2, b2):
    raise NotImplementedError("write your pallas kernel here")


# speedup vs baseline: 209.6262x; 209.6262x over previous
import functools

import jax
import jax.numpy as jnp
from jax import lax
from jax.experimental import pallas as pl
from jax.experimental.pallas import tpu as pltpu
from jax.experimental.pallas import tpu_sc as plsc

N_NODES = 50000
N_EDGES = 800000
D_NODE = 31
D_HID = 64
BOX = 6.0

NUM_CORES = 2
NUM_SUBCORES = 16
NW = NUM_CORES * NUM_SUBCORES
PER_W = N_EDGES // NW
CHUNK = 1000
N_CHUNKS = PER_W // CHUNK
TW = D_NODE + 1

BE = 8000
TC_GRID = N_EDGES // BE


def _sc_gather_body(t_hbm, rs_hbm, rr_hbm, sn_hbm, rn_hbm, idx_s, idx_r,
                    sv, rv, sem_s, sem_r):
  wid = lax.axis_index("s") * NUM_CORES + lax.axis_index("c")

  def step(c, _):
    base = pl.multiple_of(wid * PER_W + c * CHUNK, 8)
    pltpu.sync_copy(rs_hbm.at[pl.ds(base, CHUNK)], idx_s)
    pltpu.sync_copy(rr_hbm.at[pl.ds(base, CHUNK)], idx_r)
    cs = pltpu.async_copy(t_hbm.at[idx_s], sv, sem_s)
    cr = pltpu.async_copy(t_hbm.at[idx_r], rv, sem_r)
    cs.wait()
    cr.wait()
    pltpu.sync_copy(sv, sn_hbm.at[pl.ds(base, CHUNK)])
    pltpu.sync_copy(rv, rn_hbm.at[pl.ds(base, CHUNK)])
    return 0

  lax.fori_loop(0, N_CHUNKS, step, 0)


_sc_gather = functools.partial(
    pl.kernel,
    out_type=[jax.ShapeDtypeStruct((N_EDGES, TW), jnp.float32),
              jax.ShapeDtypeStruct((N_EDGES, TW), jnp.float32)],
    mesh=plsc.VectorSubcoreMesh(core_axis_name="c", subcore_axis_name="s",
                                num_cores=NUM_CORES,
                                num_subcores=NUM_SUBCORES),
    scratch_types=[
        pltpu.VMEM((CHUNK,), jnp.int32),
        pltpu.VMEM((CHUNK,), jnp.int32),
        pltpu.VMEM((CHUNK, TW), jnp.float32),
        pltpu.VMEM((CHUNK, TW), jnp.float32),
        pltpu.SemaphoreType.DMA,
        pltpu.SemaphoreType.DMA,
    ],
    compiler_params=pltpu.CompilerParams(use_tc_tiling_on_sc=False),
)(_sc_gather_body)


def _tc_mlp_body(sn_ref, rn_ref, ms_ref, mr_ref, wd_ref, b1_ref, w2t_ref,
                 b2_ref, o_ref):
  sn = sn_ref[...]
  rn = rn_ref[...]
  hi = jnp.uint32(0xFFFF0000)
  u_s = lax.bitcast_convert_type(sn[:, TW - 1:TW], jnp.uint32)
  u_r = lax.bitcast_convert_type(rn[:, TW - 1:TW], jnp.uint32)
  spx = lax.bitcast_convert_type(u_s & hi, jnp.float32)
  spy = lax.bitcast_convert_type(u_s << 16, jnp.float32)
  rpx = lax.bitcast_convert_type(u_r & hi, jnp.float32)
  rpy = lax.bitcast_convert_type(u_r << 16, jnp.float32)
  d0 = spx - rpx
  d1 = spy - rpy
  half = BOX / 2
  d0 = jnp.where(d0 > half, d0 - BOX, d0)
  d0 = jnp.where(d0 <= -half, d0 + BOX, d0)
  d1 = jnp.where(d1 > half, d1 - BOX, d1)
  d1 = jnp.where(d1 <= -half, d1 + BOX, d1)
  h = jnp.dot(sn, ms_ref[...], preferred_element_type=jnp.float32,
              precision=lax.Precision.HIGHEST)
  h = h + jnp.dot(rn, mr_ref[...], preferred_element_type=jnp.float32,
                  precision=lax.Precision.HIGHEST)
  h = h + d0 * wd_ref[0:1, :] + d1 * wd_ref[1:2, :] + b1_ref[...]
  h = jnp.maximum(h, 0.0)
  o = jnp.dot(h, w2t_ref[...], preferred_element_type=jnp.float32,
              precision=lax.Precision.HIGHEST)
  o_ref[...] = jnp.maximum(o + b2_ref[...], 0.0)


def _tc_mlp(sn, rn, ms, mr, wd, b1r, w2t, b2r):
  return pl.pallas_call(
      _tc_mlp_body,
      grid=(TC_GRID,),
      in_specs=[
          pl.BlockSpec((BE, TW), lambda i: (i, 0)),
          pl.BlockSpec((BE, TW), lambda i: (i, 0)),
          pl.BlockSpec((TW, D_HID), lambda i: (0, 0)),
          pl.BlockSpec((TW, D_HID), lambda i: (0, 0)),
          pl.BlockSpec((2, D_HID), lambda i: (0, 0)),
          pl.BlockSpec((1, D_HID), lambda i: (0, 0)),
          pl.BlockSpec((D_HID, D_HID), lambda i: (0, 0)),
          pl.BlockSpec((1, D_HID), lambda i: (0, 0)),
      ],
      out_specs=pl.BlockSpec((BE, D_HID), lambda i: (i, 0)),
      out_shape=jax.ShapeDtypeStruct((N_EDGES, D_HID), jnp.float32),
  )(sn, rn, ms, mr, wd, b1r, w2t, b2r)


@jax.jit
def kernel(V_no_pos, V_pos, R_s, R_r, W1, b1, W2, b2):
  v = V_no_pos[0]
  p = V_pos[0]
  px_bits = lax.bitcast_convert_type(
      p[:, 0].astype(jnp.bfloat16), jnp.uint16).astype(jnp.uint32)
  py_bits = lax.bitcast_convert_type(
      p[:, 1].astype(jnp.bfloat16), jnp.uint16).astype(jnp.uint32)
  packed = lax.bitcast_convert_type((px_bits << 16) | py_bits, jnp.float32)
  table = jnp.concatenate([v, packed[:, None]], axis=1)

  rs = R_s[0, :, 0].astype(jnp.int32)
  rr = R_r[0, :, 0].astype(jnp.int32)
  sn, rn = _sc_gather(table, rs, rr)

  ms = jnp.zeros((TW, D_HID), jnp.float32)
  ms = ms.at[0:D_NODE, :].set(W1[:, 0:D_NODE].T)
  mr = jnp.zeros((TW, D_HID), jnp.float32)
  mr = mr.at[0:D_NODE, :].set(W1[:, D_NODE:2 * D_NODE].T)
  wd = W1[:, 2 * D_NODE:2 * D_NODE + 2].T

  out = _tc_mlp(sn, rn, ms, mr, wd, b1[None, :], W2.T, b2[None, :])
  return out[None]


# speedup vs baseline: 384.9453x; 1.8363x over previous
import functools

import jax
import jax.numpy as jnp
from jax import lax
from jax.experimental import pallas as pl
from jax.experimental.pallas import tpu as pltpu
from jax.experimental.pallas import tpu_sc as plsc

N_NODES = 50000
N_EDGES = 800000
D_NODE = 31
D_HID = 64
BOX = 6.0

NUM_CORES = 2
NUM_SUBCORES = 16
NW = NUM_CORES * NUM_SUBCORES
PER_W = N_EDGES // NW
CHUNK = 1000
N_CHUNKS = PER_W // CHUNK
TW = D_NODE + 1

BE = 8000
TC_GRID = N_EDGES // BE


def _sc_gather_body(t_hbm, rs_hbm, rr_hbm, sn_hbm, rn_hbm, idx_s, idx_r,
                    sv, rv, sem_s, sem_r):
  wid = lax.axis_index("s") * NUM_CORES + lax.axis_index("c")

  def step(c, _):
    base = pl.multiple_of(wid * PER_W + c * CHUNK, 8)
    pltpu.sync_copy(rs_hbm.at[pl.ds(base, CHUNK)], idx_s)
    pltpu.sync_copy(rr_hbm.at[pl.ds(base, CHUNK)], idx_r)
    cs = pltpu.async_copy(t_hbm.at[idx_s], sv, sem_s)
    cr = pltpu.async_copy(t_hbm.at[idx_r], rv, sem_r)
    cs.wait()
    cr.wait()
    pltpu.sync_copy(sv, sn_hbm.at[pl.ds(base, CHUNK)])
    pltpu.sync_copy(rv, rn_hbm.at[pl.ds(base, CHUNK)])
    return 0

  lax.fori_loop(0, N_CHUNKS, step, 0)


_sc_gather = functools.partial(
    pl.kernel,
    out_type=[jax.ShapeDtypeStruct((N_EDGES, TW), jnp.float32),
              jax.ShapeDtypeStruct((N_EDGES, TW), jnp.float32)],
    mesh=plsc.VectorSubcoreMesh(core_axis_name="c", subcore_axis_name="s",
                                num_cores=NUM_CORES,
                                num_subcores=NUM_SUBCORES),
    scratch_types=[
        pltpu.VMEM((CHUNK,), jnp.int32),
        pltpu.VMEM((CHUNK,), jnp.int32),
        pltpu.VMEM((CHUNK, TW), jnp.float32),
        pltpu.VMEM((CHUNK, TW), jnp.float32),
        pltpu.SemaphoreType.DMA,
        pltpu.SemaphoreType.DMA,
    ],
    compiler_params=pltpu.CompilerParams(use_tc_tiling_on_sc=False),
)(_sc_gather_body)


def _tc_mlp_body(sn_ref, rn_ref, ms_ref, mr_ref, b1_ref, w2t_ref,
                 b2_ref, o_ref):
  sn = sn_ref[...]
  rn = rn_ref[...]
  hi = jnp.uint32(0xFFFF0000)
  u_s = lax.bitcast_convert_type(sn[:, TW - 1:TW], jnp.uint32)
  u_r = lax.bitcast_convert_type(rn[:, TW - 1:TW], jnp.uint32)
  spx = lax.bitcast_convert_type(u_s & hi, jnp.float32)
  spy = lax.bitcast_convert_type(u_s << 16, jnp.float32)
  rpx = lax.bitcast_convert_type(u_r & hi, jnp.float32)
  rpy = lax.bitcast_convert_type(u_r << 16, jnp.float32)
  d0 = spx - rpx
  d1 = spy - rpy
  half = BOX / 2
  d0 = jnp.where(d0 > half, d0 - BOX, d0)
  d0 = jnp.where(d0 <= -half, d0 + BOX, d0)
  d1 = jnp.where(d1 > half, d1 - BOX, d1)
  d1 = jnp.where(d1 <= -half, d1 + BOX, d1)
  lane = lax.broadcasted_iota(jnp.int32, (BE, TW), 1)
  sn = jnp.where(lane == TW - 1, d0, sn)
  rn = jnp.where(lane == TW - 1, d1, rn)
  h = jnp.dot(sn, ms_ref[...], preferred_element_type=jnp.float32)
  h = h + jnp.dot(rn, mr_ref[...], preferred_element_type=jnp.float32)
  h = jnp.maximum(h + b1_ref[...], 0.0)
  o = jnp.dot(h, w2t_ref[...], preferred_element_type=jnp.float32)
  o_ref[...] = jnp.maximum(o + b2_ref[...], 0.0)


def _tc_mlp(sn, rn, ms, mr, b1r, w2t, b2r):
  return pl.pallas_call(
      _tc_mlp_body,
      grid=(TC_GRID,),
      in_specs=[
          pl.BlockSpec((BE, TW), lambda i: (i, 0)),
          pl.BlockSpec((BE, TW), lambda i: (i, 0)),
          pl.BlockSpec((TW, D_HID), lambda i: (0, 0)),
          pl.BlockSpec((TW, D_HID), lambda i: (0, 0)),
          pl.BlockSpec((1, D_HID), lambda i: (0, 0)),
          pl.BlockSpec((D_HID, D_HID), lambda i: (0, 0)),
          pl.BlockSpec((1, D_HID), lambda i: (0, 0)),
      ],
      out_specs=pl.BlockSpec((BE, D_HID), lambda i: (i, 0)),
      out_shape=jax.ShapeDtypeStruct((N_EDGES, D_HID), jnp.float32),
  )(sn, rn, ms, mr, b1r, w2t, b2r)


@jax.jit
def kernel(V_no_pos, V_pos, R_s, R_r, W1, b1, W2, b2):
  v = V_no_pos[0]
  p = V_pos[0]
  px_bits = lax.bitcast_convert_type(
      p[:, 0].astype(jnp.bfloat16), jnp.uint16).astype(jnp.uint32)
  py_bits = lax.bitcast_convert_type(
      p[:, 1].astype(jnp.bfloat16), jnp.uint16).astype(jnp.uint32)
  packed = lax.bitcast_convert_type((px_bits << 16) | py_bits, jnp.float32)
  table = jnp.concatenate([v, packed[:, None]], axis=1)

  rs = R_s[0, :, 0].astype(jnp.int32)
  rr = R_r[0, :, 0].astype(jnp.int32)
  sn, rn = _sc_gather(table, rs, rr)

  ms = jnp.concatenate([W1[:, 0:D_NODE].T, W1[:, 2 * D_NODE:2 * D_NODE + 1].T],
                       axis=0)
  mr = jnp.concatenate([W1[:, D_NODE:2 * D_NODE].T,
                        W1[:, 2 * D_NODE + 1:2 * D_NODE + 2].T], axis=0)

  out = _tc_mlp(sn, rn, ms, mr, b1[None, :], W2.T, b2[None, :])
  return out[None]


# speedup vs baseline: 603.4228x; 1.5676x over previous
import functools

import jax
import jax.numpy as jnp
from jax import lax
from jax.experimental import pallas as pl
from jax.experimental.pallas import tpu as pltpu
from jax.experimental.pallas import tpu_sc as plsc

N_NODES = 50000
N_EDGES = 800000
D_NODE = 31
D_HID = 64
BOX = 6.0

NUM_CORES = 2
NUM_SUBCORES = 16
NW = NUM_CORES * NUM_SUBCORES
PER_W = N_EDGES // NW
CHUNK = 1000
N_CHUNKS = PER_W // CHUNK
TW = D_NODE + 1

BE = 8000
TC_GRID = N_EDGES // BE


def _sc_gather_body(t_hbm, rs_hbm, rr_hbm, g_hbm, idx_s, idx_r,
                    sv, rv, sem_s, sem_r):
  wid = lax.axis_index("s") * NUM_CORES + lax.axis_index("c")

  def step(c, _):
    base = pl.multiple_of(wid * PER_W + c * CHUNK, 8)
    pltpu.sync_copy(rs_hbm.at[pl.ds(base, CHUNK)], idx_s)
    pltpu.sync_copy(rr_hbm.at[pl.ds(base, CHUNK)], idx_r)
    cs = pltpu.async_copy(t_hbm.at[idx_s], sv, sem_s)
    cr = pltpu.async_copy(t_hbm.at[idx_r], rv, sem_r)
    cs.wait()
    cr.wait()
    pltpu.sync_copy(sv, g_hbm.at[pl.ds(base, CHUNK), pl.ds(0, TW)])
    pltpu.sync_copy(rv, g_hbm.at[pl.ds(base, CHUNK), pl.ds(TW, TW)])
    return 0

  lax.fori_loop(0, N_CHUNKS, step, 0)


_sc_gather = functools.partial(
    pl.kernel,
    out_type=jax.ShapeDtypeStruct((N_EDGES, 128), jnp.float32),
    mesh=plsc.VectorSubcoreMesh(core_axis_name="c", subcore_axis_name="s",
                                num_cores=NUM_CORES,
                                num_subcores=NUM_SUBCORES),
    scratch_types=[
        pltpu.VMEM((CHUNK,), jnp.int32),
        pltpu.VMEM((CHUNK,), jnp.int32),
        pltpu.VMEM((CHUNK, TW), jnp.float32),
        pltpu.VMEM((CHUNK, TW), jnp.float32),
        pltpu.SemaphoreType.DMA,
        pltpu.SemaphoreType.DMA,
    ],
    compiler_params=pltpu.CompilerParams(use_tc_tiling_on_sc=False),
)(_sc_gather_body)


def _tc_mlp_body(g_ref, m_ref, b1_ref, w2t_ref, b2_ref, o_ref):
  g = g_ref[...]
  hi = jnp.uint32(0xFFFF0000)
  u_s = lax.bitcast_convert_type(g[:, TW - 1:TW], jnp.uint32)
  u_r = lax.bitcast_convert_type(g[:, 2 * TW - 1:2 * TW], jnp.uint32)
  spx = lax.bitcast_convert_type(u_s & hi, jnp.float32)
  spy = lax.bitcast_convert_type(u_s << 16, jnp.float32)
  rpx = lax.bitcast_convert_type(u_r & hi, jnp.float32)
  rpy = lax.bitcast_convert_type(u_r << 16, jnp.float32)
  d0 = spx - rpx
  d1 = spy - rpy
  half = BOX / 2
  d0 = jnp.where(d0 > half, d0 - BOX, d0)
  d0 = jnp.where(d0 <= -half, d0 + BOX, d0)
  d1 = jnp.where(d1 > half, d1 - BOX, d1)
  d1 = jnp.where(d1 <= -half, d1 + BOX, d1)
  lane = lax.broadcasted_iota(jnp.int32, (BE, 2 * TW * 2), 1)
  g = jnp.where(lane == TW - 1, d0, g)
  g = jnp.where(lane == 2 * TW - 1, d1, g)
  g = jnp.where(lane > 2 * TW - 1, 0.0, g)
  h = jnp.dot(g, m_ref[...], preferred_element_type=jnp.float32)
  h = jnp.maximum(h + b1_ref[...], 0.0)
  o = jnp.dot(h, w2t_ref[...], preferred_element_type=jnp.float32)
  o_ref[...] = jnp.maximum(o + b2_ref[...], 0.0)


def _tc_mlp(g, m, b1r, w2t, b2r):
  return pl.pallas_call(
      _tc_mlp_body,
      grid=(TC_GRID,),
      in_specs=[
          pl.BlockSpec((BE, 128), lambda i: (i, 0)),
          pl.BlockSpec((128, D_HID), lambda i: (0, 0)),
          pl.BlockSpec((1, D_HID), lambda i: (0, 0)),
          pl.BlockSpec((D_HID, D_HID), lambda i: (0, 0)),
          pl.BlockSpec((1, D_HID), lambda i: (0, 0)),
      ],
      out_specs=pl.BlockSpec((BE, D_HID), lambda i: (i, 0)),
      out_shape=jax.ShapeDtypeStruct((N_EDGES, D_HID), jnp.float32),
  )(g, m, b1r, w2t, b2r)


@jax.jit
def kernel(V_no_pos, V_pos, R_s, R_r, W1, b1, W2, b2):
  v = V_no_pos[0]
  p = V_pos[0]
  px_bits = lax.bitcast_convert_type(
      p[:, 0].astype(jnp.bfloat16), jnp.uint16).astype(jnp.uint32)
  py_bits = lax.bitcast_convert_type(
      p[:, 1].astype(jnp.bfloat16), jnp.uint16).astype(jnp.uint32)
  packed = lax.bitcast_convert_type((px_bits << 16) | py_bits, jnp.float32)
  table = jnp.concatenate([v, packed[:, None]], axis=1)

  rs = R_s[0, :, 0].astype(jnp.int32)
  rr = R_r[0, :, 0].astype(jnp.int32)
  g = _sc_gather(table, rs, rr)

  m = jnp.concatenate(
      [W1[:, 0:D_NODE].T,
       W1[:, 2 * D_NODE:2 * D_NODE + 1].T,
       W1[:, D_NODE:2 * D_NODE].T,
       W1[:, 2 * D_NODE + 1:2 * D_NODE + 2].T,
       jnp.zeros((64, D_HID), jnp.float32)], axis=0)

  out = _tc_mlp(g, m, b1[None, :], W2.T, b2[None, :])
  return out[None]
